# Initial kernel scaffold; baseline (speedup 1.0000x reference)
import jax
import jax.numpy as jnp
from jax.experimental import pallas as pl


def kernel(x, edge_index, edge_attr, params):
    raise NotImplementedError("write your pallas kernel here")


# speedup vs baseline: 1.8241x; 1.8241x over previous
import functools

import jax
import jax.numpy as jnp
from jax import lax
from jax.experimental import pallas as pl
from jax.experimental.pallas import tpu as pltpu
from jax.experimental.pallas import tpu_sc as plsc

N = 10000
E = 320000
DN = 32
DE = 32
H = 64
ALPHA = 0.5
EPS = 1e-3

f32 = jnp.float32

NCORE = 2
NSUB = 16
NW = NCORE * NSUB
EW = E // NW
CH = 80
NCH = EW // CH
NPAD = 10240
RPS = NPAD // NSUB

BE = 8000
GE = E // BE


def _vec_mesh():
    return plsc.VectorSubcoreMesh(core_axis_name="c", subcore_axis_name="s")


_SC_PARAMS = pltpu.CompilerParams(use_tc_tiling_on_sc=False)


def _sc_gather2(h, src, dst):
    @functools.partial(
        pl.kernel,
        out_type=(
            jax.ShapeDtypeStruct((E, DN), f32),
            jax.ShapeDtypeStruct((E, DN), f32),
        ),
        mesh=_vec_mesh(),
        compiler_params=_SC_PARAMS,
        scratch_types=[
            pltpu.VMEM((CH,), jnp.int32),
            pltpu.VMEM((CH,), jnp.int32),
            pltpu.VMEM((CH, DN), f32),
            pltpu.VMEM((CH, DN), f32),
        ],
    )
    def gk(h_hbm, src_hbm, dst_hbm, hs_hbm, hd_hbm, is_v, id_v, rs_v, rd_v):
        c = lax.axis_index("c")
        s = lax.axis_index("s")
        base = (s * NCORE + c) * EW

        @pl.loop(0, NCH)
        def _(i):
            off = base + i * CH
            pltpu.sync_copy(src_hbm.at[pl.ds(off, CH)], is_v)
            pltpu.sync_copy(dst_hbm.at[pl.ds(off, CH)], id_v)
            pltpu.sync_copy(h_hbm.at[is_v], rs_v)
            pltpu.sync_copy(h_hbm.at[id_v], rd_v)
            pltpu.sync_copy(rs_v, hs_hbm.at[pl.ds(off, CH)])
            pltpu.sync_copy(rd_v, hd_hbm.at[pl.ds(off, CH)])

    return gk(h, src, dst)


def _sc_scatter_add(m, dst, zpad):
    @functools.partial(
        pl.kernel,
        out_type=jax.ShapeDtypeStruct((NCORE, NPAD, DN), f32),
        mesh=_vec_mesh(),
        compiler_params=_SC_PARAMS,
        scratch_types=[
            pltpu.VMEM((CH,), jnp.int32),
            pltpu.VMEM((CH, DN), f32),
            pltpu.VMEM_SHARED((NPAD, DN), f32),
        ],
    )
    def sk(m_hbm, dst_hbm, z_hbm, part_hbm, id_v, mb_v, agg_sh):
        c = lax.axis_index("c")
        s = lax.axis_index("s")
        base = (s * NCORE + c) * EW
        r0 = s * RPS
        pltpu.sync_copy(z_hbm.at[pl.ds(r0, RPS)], agg_sh.at[pl.ds(r0, RPS)])
        plsc.subcore_barrier()

        @pl.loop(0, NCH)
        def _(i):
            off = base + i * CH
            pltpu.sync_copy(dst_hbm.at[pl.ds(off, CH)], id_v)
            pltpu.sync_copy(m_hbm.at[pl.ds(off, CH)], mb_v)
            pltpu.sync_copy(mb_v, agg_sh.at[id_v], add=True)

        plsc.subcore_barrier()
        pltpu.sync_copy(
            agg_sh.at[pl.ds(r0, RPS)], part_hbm.at[c, pl.ds(r0, RPS)]
        )

    return sk(m, dst, zpad)


def _node_enc(x, W0, W1):
    def body(x_ref, w0_ref, w1_ref, o_ref):
        t = jnp.maximum(jnp.dot(x_ref[...], w0_ref[...],
                                preferred_element_type=f32), 0.0)
        o_ref[...] = jnp.maximum(jnp.dot(t, w1_ref[...],
                                         preferred_element_type=f32), 0.0)

    return pl.pallas_call(
        body, out_shape=jax.ShapeDtypeStruct((N, DN), f32)
    )(x, W0, W1)


def _edge_enc(ea, W0, W1):
    def body(a_ref, w0_ref, w1_ref, o_ref):
        t = jnp.maximum(jnp.dot(a_ref[...], w0_ref[...],
                                preferred_element_type=f32), 0.0)
        o_ref[...] = jnp.maximum(jnp.dot(t, w1_ref[...],
                                         preferred_element_type=f32), 0.0)

    return pl.pallas_call(
        body,
        grid=(GE,),
        in_specs=[
            pl.BlockSpec((BE, 16), lambda i: (i, 0)),
            pl.BlockSpec((16, H), lambda i: (0, 0)),
            pl.BlockSpec((H, DE), lambda i: (0, 0)),
        ],
        out_specs=pl.BlockSpec((BE, DE), lambda i: (i, 0)),
        out_shape=jax.ShapeDtypeStruct((E, DE), f32),
    )(ea, W0, W1)


def _rel_mlp(hs, hd, e, W0, b0, W1, b1):
    def body(hs_ref, hd_ref, e_ref, w0_ref, b0_ref, w1_ref, b1_ref,
             m_ref, en_ref):
        mi = jnp.concatenate([hs_ref[...], hd_ref[...], e_ref[...]], axis=1)
        t = jnp.maximum(jnp.dot(mi, w0_ref[...],
                                preferred_element_type=f32) + b0_ref[...], 0.0)
        m = jnp.maximum(jnp.dot(t, w1_ref[...],
                                preferred_element_type=f32) + b1_ref[...], 0.0)
        m_ref[...] = m
        en_ref[...] = ALPHA * e_ref[...] + (1.0 - ALPHA) * m

    return pl.pallas_call(
        body,
        grid=(GE,),
        in_specs=[
            pl.BlockSpec((BE, DN), lambda i: (i, 0)),
            pl.BlockSpec((BE, DN), lambda i: (i, 0)),
            pl.BlockSpec((BE, DE), lambda i: (i, 0)),
            pl.BlockSpec((2 * DN + DE, H), lambda i: (0, 0)),
            pl.BlockSpec((1, H), lambda i: (0, 0)),
            pl.BlockSpec((H, DE), lambda i: (0, 0)),
            pl.BlockSpec((1, DE), lambda i: (0, 0)),
        ],
        out_specs=[
            pl.BlockSpec((BE, DE), lambda i: (i, 0)),
            pl.BlockSpec((BE, DE), lambda i: (i, 0)),
        ],
        out_shape=[
            jax.ShapeDtypeStruct((E, DE), f32),
            jax.ShapeDtypeStruct((E, DE), f32),
        ],
    )(hs, hd, e, W0, b0, W1, b1)


def _obj_mlp(h, part, W0, b0, W1, b1):
    def body(h_ref, p_ref, w0_ref, b0_ref, w1_ref, b1_ref, o_ref):
        agg = p_ref[0, :N, :] + p_ref[1, :N, :]
        oi = jnp.concatenate([h_ref[...], agg], axis=1)
        t = jnp.maximum(jnp.dot(oi, w0_ref[...],
                                preferred_element_type=f32) + b0_ref[...], 0.0)
        o = jnp.maximum(jnp.dot(t, w1_ref[...],
                                preferred_element_type=f32) + b1_ref[...], 0.0)
        o_ref[...] = ALPHA * h_ref[...] + (1.0 - ALPHA) * o

    return pl.pallas_call(
        body, out_shape=jax.ShapeDtypeStruct((N, DN), f32)
    )(h, part, W0, b0, W1, b1)


def _final_mlp(hs, hd, e1, e2, e3, W0, b0, W1, b1, W2, b2):
    def body(hs_ref, hd_ref, e1_ref, e2_ref, e3_ref, w0_ref, b0_ref,
             w1_ref, b1_ref, w2_ref, b2_ref, o_ref):
        zi = jnp.concatenate(
            [hs_ref[...], hd_ref[...], e1_ref[...], e2_ref[...], e3_ref[...]],
            axis=1)
        z = jnp.maximum(jnp.dot(zi, w0_ref[...],
                                preferred_element_type=f32) + b0_ref[...], 0.0)
        z = jnp.maximum(jnp.dot(z, w1_ref[...],
                                preferred_element_type=f32) + b1_ref[...], 0.0)
        logit = jnp.dot(z, w2_ref[...],
                        preferred_element_type=f32) + b2_ref[...]
        o_ref[...] = EPS + (1.0 - 2.0 * EPS) * jax.nn.sigmoid(logit)

    WIN = 3 * DE + 2 * DN
    return pl.pallas_call(
        body,
        grid=(GE,),
        in_specs=[
            pl.BlockSpec((BE, DN), lambda i: (i, 0)),
            pl.BlockSpec((BE, DN), lambda i: (i, 0)),
            pl.BlockSpec((BE, DE), lambda i: (i, 0)),
            pl.BlockSpec((BE, DE), lambda i: (i, 0)),
            pl.BlockSpec((BE, DE), lambda i: (i, 0)),
            pl.BlockSpec((WIN, H), lambda i: (0, 0)),
            pl.BlockSpec((1, H), lambda i: (0, 0)),
            pl.BlockSpec((H, H), lambda i: (0, 0)),
            pl.BlockSpec((1, H), lambda i: (0, 0)),
            pl.BlockSpec((H, 1), lambda i: (0, 0)),
            pl.BlockSpec((1, 1), lambda i: (0, 0)),
        ],
        out_specs=pl.BlockSpec((BE, 1), lambda i: (i, 0)),
        out_shape=jax.ShapeDtypeStruct((E, 1), f32),
    )(hs, hd, e1, e2, e3, W0, b0, W1, b1, W2, b2)


def kernel(x, edge_index, edge_attr, params):
    src = edge_index[0]
    dst = edge_index[1]

    h = _node_enc(x, params["node_enc"]["W0"], params["node_enc"]["W1"])
    e = _edge_enc(edge_attr, params["edge_enc"]["W0"],
                  params["edge_enc"]["W1"])
    zpad = jnp.zeros((NPAD, DN), f32)

    e_embeds = []
    for lp in params["resin"]:
        hs, hd = _sc_gather2(h, src, dst)
        m, e = _rel_mlp(hs, hd, e,
                        lp["rel_W0"], lp["rel_b0"].reshape(1, H),
                        lp["rel_W1"], lp["rel_b1"].reshape(1, DE))
        part = _sc_scatter_add(m, dst, zpad)
        h = _obj_mlp(h, part,
                     lp["obj_W0"], lp["obj_b0"].reshape(1, H),
                     lp["obj_W1"], lp["obj_b1"].reshape(1, DN))
        e_embeds.append(e)

    hs, hd = _sc_gather2(h, src, dst)
    wm = params["W_mlp"]
    w2d = _final_mlp(hs, hd, e_embeds[0], e_embeds[1], e_embeds[2],
                     wm["W0"], wm["b0"].reshape(1, H),
                     wm["W1"], wm["b1"].reshape(1, H),
                     wm["W2"], wm["b2"].reshape(1, 1))
    return (w2d[:, 0], h, e_embeds[2])


# speedup vs baseline: 5.5596x; 3.0478x over previous
import functools

import jax
import jax.numpy as jnp
from jax import lax
from jax.experimental import pallas as pl
from jax.experimental.pallas import tpu as pltpu
from jax.experimental.pallas import tpu_sc as plsc

N = 10000
E = 320000
DN = 32
DE = 32
H = 64
ALPHA = 0.5
EPS = 1e-3

f32 = jnp.float32

NCORE = 2
NSUB = 16
NW = NCORE * NSUB
EW = E // NW
CH = 80
NCH = EW // CH
NPAD = 10240
RPS = NPAD // NSUB

BLK = 640
NBLK = 16
_BLK_NCH = [8] * 15 + [5]

BE = 8000
GE = E // BE
B4 = BE // 4
E4 = E // 4


def _vec_mesh():
    return plsc.VectorSubcoreMesh(core_axis_name="c", subcore_axis_name="s")


_SC_PARAMS = pltpu.CompilerParams(use_tc_tiling_on_sc=False)


def _sc_gather2(h, src2d, dst2d):
    @functools.partial(
        pl.kernel,
        out_type=(
            jax.ShapeDtypeStruct((E, DN), f32),
            jax.ShapeDtypeStruct((E, DN), f32),
        ),
        mesh=_vec_mesh(),
        compiler_params=_SC_PARAMS,
        scratch_types=[
            pltpu.VMEM((NCH, CH), jnp.int32),
            pltpu.VMEM((NCH, CH), jnp.int32),
            pltpu.VMEM((BLK, DN), f32),
            pltpu.VMEM((BLK, DN), f32),
            pltpu.VMEM((BLK, DN), f32),
            pltpu.VMEM((BLK, DN), f32),
            pltpu.SemaphoreType.DMA,
            pltpu.SemaphoreType.DMA,
            pltpu.SemaphoreType.DMA,
            pltpu.SemaphoreType.DMA,
            pltpu.SemaphoreType.DMA,
            pltpu.SemaphoreType.DMA,
        ],
    )
    def gk(h_hbm, src_hbm, dst_hbm, hs_hbm, hd_hbm,
           is2, id2, rs0, rs1, rd0, rd1,
           gs_sem, gd_sem, ws0_sem, ws1_sem, wd0_sem, wd1_sem):
        c = lax.axis_index("c")
        s = lax.axis_index("s")
        wid = s * NCORE + c
        base = wid * EW
        base_r = wid * NCH
        pltpu.sync_copy(src_hbm.at[pl.ds(base_r, NCH)], is2)
        pltpu.sync_copy(dst_hbm.at[pl.ds(base_r, NCH)], id2)

        rs = [rs0, rs1]
        rd = [rd0, rd1]
        wsem = [(ws0_sem, wd0_sem), (ws1_sem, wd1_sem)]

        def issue_block(b, cur):
            for j in range(_BLK_NCH[b]):
                r = b * 8 + j
                pltpu.async_copy(h_hbm.at[is2.at[r]],
                                 rs[cur].at[pl.ds(j * CH, CH)], gs_sem)
                pltpu.async_copy(h_hbm.at[id2.at[r]],
                                 rd[cur].at[pl.ds(j * CH, CH)], gd_sem)

        wh = [None, None]
        issue_block(0, 0)

        for b in range(NBLK):
            cur = b & 1
            nxt = 1 - cur
            n = _BLK_NCH[b] * CH
            off = base + b * BLK
            for _ in range(_BLK_NCH[b]):
                pltpu.make_async_copy(
                    h_hbm.at[is2.at[0]], rs[cur].at[pl.ds(0, CH)], gs_sem
                ).wait()
                pltpu.make_async_copy(
                    h_hbm.at[id2.at[0]], rd[cur].at[pl.ds(0, CH)], gd_sem
                ).wait()
            ws = pltpu.async_copy(rs[cur].at[pl.ds(0, n)],
                                  hs_hbm.at[pl.ds(off, n)], wsem[cur][0])
            wd = pltpu.async_copy(rd[cur].at[pl.ds(0, n)],
                                  hd_hbm.at[pl.ds(off, n)], wsem[cur][1])
            wh[cur] = (ws, wd)
            if b + 1 < NBLK:
                if wh[nxt] is not None:
                    wh[nxt][0].wait()
                    wh[nxt][1].wait()
                issue_block(b + 1, nxt)
        wh[0][0].wait()
        wh[0][1].wait()
        wh[1][0].wait()
        wh[1][1].wait()

    return gk(h, src2d, dst2d)


def _sc_scatter_add(m, dst2d, zpad):
    @functools.partial(
        pl.kernel,
        out_type=jax.ShapeDtypeStruct((NCORE, NPAD, DN), f32),
        mesh=_vec_mesh(),
        compiler_params=_SC_PARAMS,
        scratch_types=[
            pltpu.VMEM((NCH, CH), jnp.int32),
            pltpu.VMEM((BLK, DN), f32),
            pltpu.VMEM((BLK, DN), f32),
            pltpu.VMEM_SHARED((NPAD, DN), f32),
            pltpu.SemaphoreType.DMA,
            pltpu.SemaphoreType.DMA,
            pltpu.SemaphoreType.DMA,
            pltpu.SemaphoreType.DMA,
        ],
    )
    def sk(m_hbm, dst_hbm, z_hbm, part_hbm, id2, mb0, mb1, agg_sh,
           l0_sem, l1_sem, s0_sem, s1_sem):
        c = lax.axis_index("c")
        s = lax.axis_index("s")
        wid = s * NCORE + c
        base = wid * EW
        base_r = wid * NCH
        r0 = s * RPS
        pltpu.sync_copy(z_hbm.at[pl.ds(r0, RPS)], agg_sh.at[pl.ds(r0, RPS)])
        pltpu.sync_copy(dst_hbm.at[pl.ds(base_r, NCH)], id2)
        plsc.subcore_barrier()

        mb = [mb0, mb1]
        lsem = [l0_sem, l1_sem]
        ssem = [s0_sem, s1_sem]
        lh = [None, None]
        sh_n = [0, 0]

        def issue_load(b, cur):
            n = _BLK_NCH[b] * CH
            lh[cur] = pltpu.async_copy(
                m_hbm.at[pl.ds(base + b * BLK, n)],
                mb[cur].at[pl.ds(0, n)], lsem[cur])

        issue_load(0, 0)
        for b in range(NBLK):
            cur = b & 1
            nxt = 1 - cur
            lh[cur].wait()
            if b + 1 < NBLK:
                for _ in range(sh_n[nxt]):
                    pltpu.make_async_copy(
                        mb[nxt].at[pl.ds(0, CH)],
                        agg_sh.at[pl.ds(0, CH)], ssem[nxt]
                    ).wait()
                sh_n[nxt] = 0
                issue_load(b + 1, nxt)
            for j in range(_BLK_NCH[b]):
                pltpu.async_copy(mb[cur].at[pl.ds(j * CH, CH)],
                                 agg_sh.at[id2.at[b * 8 + j]],
                                 ssem[cur], add=True)
            sh_n[cur] = _BLK_NCH[b]
        for par in (0, 1):
            for _ in range(sh_n[par]):
                pltpu.make_async_copy(
                    mb[par].at[pl.ds(0, CH)],
                    agg_sh.at[pl.ds(0, CH)], ssem[par]
                ).wait()

        plsc.subcore_barrier()
        pltpu.sync_copy(
            agg_sh.at[pl.ds(r0, RPS)], part_hbm.at[c, pl.ds(r0, RPS)]
        )

    return sk(m, dst2d, zpad)


def _node_enc(x, W0, W1):
    def body(x_ref, w0_ref, w1_ref, o_ref):
        t = jnp.maximum(jnp.dot(x_ref[...], w0_ref[...],
                                preferred_element_type=f32), 0.0)
        o_ref[...] = jnp.maximum(jnp.dot(t, w1_ref[...],
                                         preferred_element_type=f32), 0.0)

    return pl.pallas_call(
        body, out_shape=jax.ShapeDtypeStruct((N, DN), f32)
    )(x, W0, W1)


def _edge_enc(ea, W0, W1):
    def body(a0_ref, a1_ref, a2_ref, a3_ref, w0_ref, w1_ref, o_ref):
        outs = []
        for a_ref in (a0_ref, a1_ref, a2_ref, a3_ref):
            t = jnp.maximum(jnp.dot(a_ref[...], w0_ref[...],
                                    preferred_element_type=f32), 0.0)
            outs.append(jnp.maximum(jnp.dot(t, w1_ref[...],
                                            preferred_element_type=f32), 0.0))
        o_ref[...] = jnp.concatenate(outs, axis=1)

    return pl.pallas_call(
        body,
        grid=(GE,),
        in_specs=[
            pl.BlockSpec((B4, 16), lambda i, g=g: (GE * g + i, 0))
            for g in range(4)
        ] + [
            pl.BlockSpec((16, H), lambda i: (0, 0)),
            pl.BlockSpec((H, DE), lambda i: (0, 0)),
        ],
        out_specs=pl.BlockSpec((B4, 128), lambda i: (i, 0)),
        out_shape=jax.ShapeDtypeStruct((E4, 128), f32),
    )(ea, ea, ea, ea, W0, W1)


def _rel_mlp(hs, hd, e, W0, b0, W1, b1):
    def body(hs_ref, hd_ref, e_ref, w0_ref, b0_ref, w1_ref, b1_ref,
             m_ref, en_ref):
        hs = hs_ref[...]
        hd = hd_ref[...]
        ea = e_ref[...]
        ms = []
        for g in range(4):
            sl = slice(DN * g, DN * (g + 1))
            mi = jnp.concatenate([hs[:, sl], hd[:, sl], ea[:, sl]], axis=1)
            t = jnp.maximum(jnp.dot(mi, w0_ref[...],
                                    preferred_element_type=f32)
                            + b0_ref[...], 0.0)
            ms.append(jnp.maximum(jnp.dot(t, w1_ref[...],
                                          preferred_element_type=f32)
                                  + b1_ref[...], 0.0))
        m = jnp.concatenate(ms, axis=1)
        m_ref[...] = m
        en_ref[...] = ALPHA * ea + (1.0 - ALPHA) * m

    return pl.pallas_call(
        body,
        grid=(GE,),
        in_specs=[
            pl.BlockSpec((B4, 128), lambda i: (i, 0)),
            pl.BlockSpec((B4, 128), lambda i: (i, 0)),
            pl.BlockSpec((B4, 128), lambda i: (i, 0)),
            pl.BlockSpec((2 * DN + DE, H), lambda i: (0, 0)),
            pl.BlockSpec((1, H), lambda i: (0, 0)),
            pl.BlockSpec((H, DE), lambda i: (0, 0)),
            pl.BlockSpec((1, DE), lambda i: (0, 0)),
        ],
        out_specs=[
            pl.BlockSpec((B4, 128), lambda i: (i, 0)),
            pl.BlockSpec((B4, 128), lambda i: (i, 0)),
        ],
        out_shape=[
            jax.ShapeDtypeStruct((E4, 128), f32),
            jax.ShapeDtypeStruct((E4, 128), f32),
        ],
    )(hs, hd, e, W0, b0, W1, b1)


def _obj_mlp(h, part, W0, b0, W1, b1):
    def body(h_ref, p_ref, w0_ref, b0_ref, w1_ref, b1_ref, o_ref):
        agg = p_ref[0, :N, :] + p_ref[1, :N, :]
        oi = jnp.concatenate([h_ref[...], agg], axis=1)
        t = jnp.maximum(jnp.dot(oi, w0_ref[...],
                                preferred_element_type=f32) + b0_ref[...], 0.0)
        o = jnp.maximum(jnp.dot(t, w1_ref[...],
                                preferred_element_type=f32) + b1_ref[...], 0.0)
        o_ref[...] = ALPHA * h_ref[...] + (1.0 - ALPHA) * o

    return pl.pallas_call(
        body, out_shape=jax.ShapeDtypeStruct((N, DN), f32)
    )(h, part, W0, b0, W1, b1)


def _final_mlp(hs, hd, e1, e2, e3, W0, b0, W1, b1, W2, b2):
    def body(hs_ref, hd_ref, e1_ref, e2_ref, e3_ref, w0_ref, b0_ref,
             w1_ref, b1_ref, w2_ref, b2_ref, o_ref):
        hs = hs_ref[...]
        hd = hd_ref[...]
        e1 = e1_ref[...]
        e2 = e2_ref[...]
        e3 = e3_ref[...]
        outs = []
        for g in range(4):
            sl = slice(DN * g, DN * (g + 1))
            zi = jnp.concatenate(
                [hs[:, sl], hd[:, sl], e1[:, sl], e2[:, sl], e3[:, sl]],
                axis=1)
            z = jnp.maximum(jnp.dot(zi, w0_ref[...],
                                    preferred_element_type=f32)
                            + b0_ref[...], 0.0)
            z = jnp.maximum(jnp.dot(z, w1_ref[...],
                                    preferred_element_type=f32)
                            + b1_ref[...], 0.0)
            logit = jnp.dot(z, w2_ref[...],
                            preferred_element_type=f32) + b2_ref[...]
            outs.append(EPS + (1.0 - 2.0 * EPS) * jax.nn.sigmoid(logit))
        o_ref[...] = jnp.concatenate(outs, axis=1)

    WIN = 3 * DE + 2 * DN
    return pl.pallas_call(
        body,
        grid=(GE,),
        in_specs=[
            pl.BlockSpec((B4, 128), lambda i: (i, 0)),
            pl.BlockSpec((B4, 128), lambda i: (i, 0)),
            pl.BlockSpec((B4, 128), lambda i: (i, 0)),
            pl.BlockSpec((B4, 128), lambda i: (i, 0)),
            pl.BlockSpec((B4, 128), lambda i: (i, 0)),
            pl.BlockSpec((WIN, H), lambda i: (0, 0)),
            pl.BlockSpec((1, H), lambda i: (0, 0)),
            pl.BlockSpec((H, H), lambda i: (0, 0)),
            pl.BlockSpec((1, H), lambda i: (0, 0)),
            pl.BlockSpec((H, 1), lambda i: (0, 0)),
            pl.BlockSpec((1, 1), lambda i: (0, 0)),
        ],
        out_specs=pl.BlockSpec((B4, 4), lambda i: (i, 0)),
        out_shape=jax.ShapeDtypeStruct((E4, 4), f32),
    )(hs, hd, e1, e2, e3, W0, b0, W1, b1, W2, b2)


def kernel(x, edge_index, edge_attr, params):
    src2d = edge_index[0].reshape(4, E4).T.reshape(E // CH, CH)
    dst2d = edge_index[1].reshape(4, E4).T.reshape(E // CH, CH)

    h = _node_enc(x, params["node_enc"]["W0"], params["node_enc"]["W1"])
    e = _edge_enc(edge_attr, params["edge_enc"]["W0"],
                  params["edge_enc"]["W1"])
    zpad = jnp.zeros((NPAD, DN), f32)

    e_embeds = []
    for lp in params["resin"]:
        hs, hd = _sc_gather2(h, src2d, dst2d)
        m, e = _rel_mlp(hs.reshape(E4, 128), hd.reshape(E4, 128), e,
                        lp["rel_W0"], lp["rel_b0"].reshape(1, H),
                        lp["rel_W1"], lp["rel_b1"].reshape(1, DE))
        part = _sc_scatter_add(m.reshape(E, DE), dst2d, zpad)
        h = _obj_mlp(h, part,
                     lp["obj_W0"], lp["obj_b0"].reshape(1, H),
                     lp["obj_W1"], lp["obj_b1"].reshape(1, DN))
        e_embeds.append(e)

    hs, hd = _sc_gather2(h, src2d, dst2d)
    wm = params["W_mlp"]
    w4 = _final_mlp(hs.reshape(E4, 128), hd.reshape(E4, 128),
                    e_embeds[0], e_embeds[1], e_embeds[2],
                    wm["W0"], wm["b0"].reshape(1, H),
                    wm["W1"], wm["b1"].reshape(1, H),
                    wm["W2"], wm["b2"].reshape(1, 1))
    w4n = w4.T.reshape(E)
    e3n = e_embeds[2].reshape(E4, 4, DE).transpose(1, 0, 2).reshape(E, DE)
    return (w4n, h, e3n)
